# baseline (speedup 1.0000x reference)
import functools

import jax
import jax.numpy as jnp
from jax import lax
from jax.experimental import pallas as pl
from jax.experimental.pallas import tpu as pltpu

N = 10000
N_PAD = 10240
E = 320000
E_PAD = 323584
G = 8
EMB = 128
TBL = 144
BN = 1024
BE = 1024


def _dotT(x, w):
    return lax.dot_general(x, w, (((1,), (1,)), ((), ())),
                           preferred_element_type=jnp.float32,
                           precision=lax.Precision.HIGHEST)


def _silu(x):
    return x * jax.nn.sigmoid(x)


def _artanh(x):
    x = jnp.clip(x, -1.0 + 1e-7, 1.0 - 1e-7)
    return 0.5 * (jnp.log1p(x) - jnp.log1p(-x))


def _rownorm(x):
    return jnp.clip(jnp.sqrt(jnp.sum(x * x, axis=-1, keepdims=True)), 1e-15)


def _prep_body(time_ref, tm0w, tm0b, tm1w, tm1b, tp0w, tp0b, tp1w, tp1b,
               wenc, benc, w1d0, b10, wu0, bu0, w1d1, b11,
               temb0_o, temb1_o, weffT0_o, beff0_o, weffT1_o, beff1_o):
    t1 = _silu(_dotT(time_ref[...], tm0w[...]) + tm0b[...])
    t = _dotT(t1, tm1w[...]) + tm1b[...]
    st = _silu(t)
    temb0_o[...] = _dotT(st, tp0w[...]) + tp0b[...]
    temb1_o[...] = _dotT(st, tp1w[...]) + tp1b[...]
    weffT0_o[...] = jnp.dot(w1d0[...], wenc[...],
                            preferred_element_type=jnp.float32,
                            precision=lax.Precision.HIGHEST)
    beff0_o[...] = _dotT(benc[...], w1d0[...]) + b10[...]
    weffT1_o[...] = jnp.dot(w1d1[...], wu0[...],
                            preferred_element_type=jnp.float32,
                            precision=lax.Precision.HIGHEST)
    beff1_o[...] = _dotT(bu0[...], w1d1[...]) + b11[...]


def _pre_body(x_ref, ex_ref, pos_ref, hypw, w1a, w1b,
              feats_o, tabr_o, tabc_o):
    xc = jnp.concatenate([x_ref[...], ex_ref[...]], axis=-1)
    un = _rownorm(xc)
    u1 = jnp.tanh(un) * xc / un
    n = _rownorm(u1)
    maxn = 1.0 - 4e-3
    xh = jnp.where(n > maxn, u1 / n * maxn, u1)
    xn = _rownorm(xh)
    mx = _dotT(xh, hypw[...])
    mxn = _rownorm(mx)
    hv = jnp.tanh(mxn / xn * _artanh(xn)) * mx / mxn
    n2 = _rownorm(hv)
    hv = jnp.where(n2 > maxn, hv / n2 * maxn, hv)
    pn = _rownorm(hv)
    h = _artanh(pn) * hv / pn
    feats_o[...] = h
    pos = pos_ref[...]
    z = jnp.zeros((h.shape[0], TBL - EMB - 3), jnp.float32)
    tabr_o[...] = jnp.concatenate([_dotT(h, w1a[...]), pos, z], axis=-1)
    tabc_o[...] = jnp.concatenate([_dotT(h, w1b[...]), pos, z], axis=-1)


def _edge_body(gsum_ref, s_ref, wefft, beff, w2, b2, m2_o):
    m1 = gsum_ref[...] + _dotT(s_ref[...], wefft[...]) + beff[...]
    m = _silu(m1)
    m2_o[...] = _silu(_dotT(m, w2[...]) + b2[...])


def _node_body(feats_ref, agga_ref, aggb_ref, coors_ref, batch_ref, temb,
               wn0a, wn0b, bn0, wn1, bn1, wl1, bl1, lng, lnb, wl2, bl2,
               *rest, final):
    feats = feats_ref[...]
    agg = agga_ref[...] + aggb_ref[...]
    hm = _silu(_dotT(feats, wn0a[...]) + _dotT(agg, wn0b[...]) + bn0[...])
    h2 = _dotT(hm, wn1[...]) + bn1[...]
    f_out = feats + h2
    coors = coors_ref[...]
    nsq = (jnp.sum(coors * coors, axis=-1, keepdims=True)
           + jnp.sum(f_out * f_out, axis=-1, keepdims=True))
    un = jnp.clip(jnp.sqrt(nsq), 1e-15)
    fac = jnp.tanh(un) / un
    corr = coors * fac
    f2 = f_out * fac
    iot = lax.broadcasted_iota(jnp.int32, (f2.shape[0], G), 1)
    oh = (batch_ref[...] == iot).astype(jnp.float32)
    st = jnp.dot(oh, temb[...], preferred_element_type=jnp.float32,
                 precision=lax.Precision.HIGHEST)
    scale = st[:, :EMB]
    shift = st[:, EMB:]
    f3 = f2 * (scale + 1.0) + shift
    f = _silu(_dotT(f3, wl1[...]) + bl1[...])
    mu = jnp.mean(f, axis=-1, keepdims=True)
    var = jnp.mean((f - mu) * (f - mu), axis=-1, keepdims=True)
    fn = (f - mu) / jnp.sqrt(var + 1e-5) * lng[...] + lnb[...]
    f4 = _dotT(fn, wl2[...]) + bl2[...]
    if final:
        wlin, blin, out_o = rest
        out_o[...] = _dotT(f4, wlin[...]) + blin[...]
    else:
        w1a, w1b, feats_o, corr_o, tabr_o, tabc_o = rest
        feats_o[...] = f4
        corr_o[...] = corr
        z = jnp.zeros((f4.shape[0], TBL - EMB - 3), jnp.float32)
        tabr_o[...] = jnp.concatenate([_dotT(f4, w1a[...]), corr, z], axis=-1)
        tabc_o[...] = jnp.concatenate([_dotT(f4, w1b[...]), corr, z], axis=-1)


def _full(a):
    return pl.BlockSpec(a.shape, lambda i: (0,) * a.ndim)


def _blk(shape):
    return pl.BlockSpec(shape, lambda i: (i,) + (0,) * (len(shape) - 1))


def _pad_rows(a, n):
    return jnp.pad(a, ((0, n - a.shape[0]),) + ((0, 0),) * (a.ndim - 1))


def _gather_gsum(tabr, tabc, row, col, w1c):
    gr = tabr[row]
    gc = tabc[col]
    d = gr[:, EMB:] - gc[:, EMB:]
    rel = jnp.sum(d * d, axis=-1, keepdims=True)
    return gr[:, :EMB] + gc[:, :EMB] + rel * w1c[None, :]


def _segsum(m2, row):
    agg = jax.ops.segment_sum(m2, row, num_segments=N_PAD)
    return agg, jnp.zeros_like(agg)


def kernel(x, pos, extra_x, edge_index, edge_attr, batch, time, params):
    f32 = jnp.float32
    lyr = params["layers"]
    w1 = [l["edge_mlp"][0]["w"] for l in lyr]
    w1a = [w[:, :EMB] for w in w1]
    w1b = [w[:, EMB:2 * EMB] for w in w1]
    w1c = [w[:, 2 * EMB] for w in w1]
    w1d = [w[:, 2 * EMB + 1:] for w in w1]
    b1 = [l["edge_mlp"][0]["b"].reshape(1, EMB) for l in lyr]
    w2 = [l["edge_mlp"][1]["w"] for l in lyr]
    b2 = [l["edge_mlp"][1]["b"].reshape(1, EMB) for l in lyr]
    wn0 = [l["node_mlp"][0]["w"] for l in lyr]
    bn0 = [l["node_mlp"][0]["b"].reshape(1, EMB) for l in lyr]
    wn1 = [l["node_mlp"][1]["w"] for l in lyr]
    bn1 = [l["node_mlp"][1]["b"].reshape(1, EMB) for l in lyr]
    wl1 = [l["ff"]["l1"]["w"] for l in lyr]
    bl1 = [l["ff"]["l1"]["b"].reshape(1, EMB) for l in lyr]
    lng = [l["ff"]["ln_g"].reshape(1, EMB) for l in lyr]
    lnb = [l["ff"]["ln_b"].reshape(1, EMB) for l in lyr]
    wl2 = [l["ff"]["l2"]["w"] for l in lyr]
    bl2 = [l["ff"]["l2"]["b"].reshape(1, EMB) for l in lyr]

    prep_out = pl.pallas_call(
        _prep_body,
        out_shape=[
            jax.ShapeDtypeStruct((G, 2 * EMB), f32),
            jax.ShapeDtypeStruct((G, 2 * EMB), f32),
            jax.ShapeDtypeStruct((EMB, 4), f32),
            jax.ShapeDtypeStruct((1, EMB), f32),
            jax.ShapeDtypeStruct((EMB, EMB), f32),
            jax.ShapeDtypeStruct((1, EMB), f32),
        ],
    )(time,
      params["time_mlp"][0]["w"], params["time_mlp"][0]["b"].reshape(1, -1),
      params["time_mlp"][1]["w"], params["time_mlp"][1]["b"].reshape(1, -1),
      lyr[0]["time_proj"]["w"], lyr[0]["time_proj"]["b"].reshape(1, -1),
      lyr[1]["time_proj"]["w"], lyr[1]["time_proj"]["b"].reshape(1, -1),
      params["edge_enc"]["w"], params["edge_enc"]["b"].reshape(1, -1),
      w1d[0], b1[0],
      lyr[0]["edge_upd"]["w"], lyr[0]["edge_upd"]["b"].reshape(1, -1),
      w1d[1], b1[1])
    temb0, temb1, wefft0, beff0, wefft1, beff1 = prep_out

    x_p = _pad_rows(x, N_PAD)
    ex_p = _pad_rows(extra_x, N_PAD)
    pos_p = _pad_rows(pos, N_PAD)
    batch_p = _pad_rows(batch.reshape(N, 1), N_PAD)
    npad = E_PAD - E
    fill = N + (jnp.arange(npad, dtype=jnp.int32) % (N_PAD - N))
    row_p = jnp.concatenate([edge_index[0], fill])
    col_p = jnp.concatenate([edge_index[1], fill])
    eattr_p = _pad_rows(edge_attr, E_PAD)

    grid_n = (N_PAD // BN,)
    feats0, tabr0, tabc0 = pl.pallas_call(
        _pre_body,
        grid=grid_n,
        in_specs=[_blk((BN, 12)), _blk((BN, 3)), _blk((BN, 3)),
                  _full(params["hyp_w"]), _full(w1a[0]), _full(w1b[0])],
        out_specs=[_blk((BN, EMB)), _blk((BN, TBL)), _blk((BN, TBL))],
        out_shape=[
            jax.ShapeDtypeStruct((N_PAD, EMB), f32),
            jax.ShapeDtypeStruct((N_PAD, TBL), f32),
            jax.ShapeDtypeStruct((N_PAD, TBL), f32),
        ],
    )(x_p, ex_p, pos_p, params["hyp_w"], w1a[0], w1b[0])

    grid_e = (E_PAD // BE,)

    def edge_mlp(gsum, s_e, wefft, beff, w2_, b2_):
        ds = s_e.shape[1]
        return pl.pallas_call(
            _edge_body,
            grid=grid_e,
            in_specs=[_blk((BE, EMB)), _blk((BE, ds)), _full(wefft),
                      _full(beff), _full(w2_), _full(b2_)],
            out_specs=_blk((BE, EMB)),
            out_shape=jax.ShapeDtypeStruct((E_PAD, EMB), f32),
        )(gsum, s_e, wefft, beff, w2_, b2_)

    def node_update(li, feats, agga, aggb, coors, temb, final):
        body = functools.partial(_node_body, final=final)
        common = [feats, agga, aggb, coors, batch_p, temb,
                  wn0[li][:, :EMB], wn0[li][:, EMB:], bn0[li], wn1[li],
                  bn1[li], wl1[li], bl1[li], lng[li], lnb[li], wl2[li],
                  bl2[li]]
        common_specs = [_blk((BN, EMB)), _blk((BN, EMB)), _blk((BN, EMB)),
                        _blk((BN, 3)), _blk((BN, 1)), _full(temb)] + \
            [_full(a) for a in common[6:]]
        if final:
            wlin = params["lin"]["w"]
            blin = params["lin"]["b"].reshape(1, -1)
            return pl.pallas_call(
                body, grid=grid_n,
                in_specs=common_specs + [_full(wlin), _full(blin)],
                out_specs=_blk((BN, 4)),
                out_shape=jax.ShapeDtypeStruct((N_PAD, 4), f32),
            )(*common, wlin, blin)
        return pl.pallas_call(
            body, grid=grid_n,
            in_specs=common_specs + [_full(w1a[li + 1]), _full(w1b[li + 1])],
            out_specs=[_blk((BN, EMB)), _blk((BN, 3)), _blk((BN, TBL)),
                       _blk((BN, TBL))],
            out_shape=[
                jax.ShapeDtypeStruct((N_PAD, EMB), f32),
                jax.ShapeDtypeStruct((N_PAD, 3), f32),
                jax.ShapeDtypeStruct((N_PAD, TBL), f32),
                jax.ShapeDtypeStruct((N_PAD, TBL), f32),
            ])(*common, w1a[li + 1], w1b[li + 1])

    gsum0 = _gather_gsum(tabr0, tabc0, row_p, col_p, w1c[0])
    m2_0 = edge_mlp(gsum0, eattr_p, wefft0, beff0, w2[0], b2[0])
    agg0a, agg0b = _segsum(m2_0, row_p)
    feats1, corr1, tabr1, tabc1 = node_update(
        0, feats0, agg0a, agg0b, pos_p, temb0, final=False)

    gsum1 = _gather_gsum(tabr1, tabc1, row_p, col_p, w1c[1])
    m2_1 = edge_mlp(gsum1, m2_0, wefft1, beff1, w2[1], b2[1])
    agg1a, agg1b = _segsum(m2_1, row_p)
    out_p = node_update(1, feats1, agg1a, agg1b, corr1, temb1, final=True)

    return out_p[:N]


# speedup vs baseline: 2.1124x; 2.1124x over previous
import functools

import jax
import jax.numpy as jnp
from jax import lax
from jax.experimental import pallas as pl
from jax.experimental.pallas import tpu as pltpu
from jax.experimental.pallas import tpu_sc as plsc

N = 10000
N_PAD = 10240
E = 320000
E_PAD = 323584
G = 8
EMB = 128
TBL = 256
BN = 1024
BE = 1024


def _dotT(x, w):
    return lax.dot_general(x, w, (((1,), (1,)), ((), ())),
                           preferred_element_type=jnp.float32,
                           precision=lax.Precision.HIGHEST)


def _silu(x):
    return x * jax.nn.sigmoid(x)


def _artanh(x):
    x = jnp.clip(x, -1.0 + 1e-7, 1.0 - 1e-7)
    return 0.5 * (jnp.log1p(x) - jnp.log1p(-x))


def _rownorm(x):
    return jnp.clip(jnp.sqrt(jnp.sum(x * x, axis=-1, keepdims=True)), 1e-15)


def _prep_body(time_ref, tm0w, tm0b, tm1w, tm1b, tp0w, tp0b, tp1w, tp1b,
               wenc, benc, w1d0, b10, wu0, bu0, w1d1, b11,
               temb0_o, temb1_o, weffT0_o, beff0_o, weffT1_o, beff1_o):
    t1 = _silu(_dotT(time_ref[...], tm0w[...]) + tm0b[...])
    t = _dotT(t1, tm1w[...]) + tm1b[...]
    st = _silu(t)
    temb0_o[...] = _dotT(st, tp0w[...]) + tp0b[...]
    temb1_o[...] = _dotT(st, tp1w[...]) + tp1b[...]
    weffT0_o[...] = jnp.dot(w1d0[...], wenc[...],
                            preferred_element_type=jnp.float32,
                            precision=lax.Precision.HIGHEST)
    beff0_o[...] = _dotT(benc[...], w1d0[...]) + b10[...]
    weffT1_o[...] = jnp.dot(w1d1[...], wu0[...],
                            preferred_element_type=jnp.float32,
                            precision=lax.Precision.HIGHEST)
    beff1_o[...] = _dotT(bu0[...], w1d1[...]) + b11[...]


def _pre_body(x_ref, ex_ref, pos_ref, hypw, w1a, w1b,
              feats_o, tabr_o, tabc_o):
    xc = jnp.concatenate([x_ref[...], ex_ref[...]], axis=-1)
    un = _rownorm(xc)
    u1 = jnp.tanh(un) * xc / un
    n = _rownorm(u1)
    maxn = 1.0 - 4e-3
    xh = jnp.where(n > maxn, u1 / n * maxn, u1)
    xn = _rownorm(xh)
    mx = _dotT(xh, hypw[...])
    mxn = _rownorm(mx)
    hv = jnp.tanh(mxn / xn * _artanh(xn)) * mx / mxn
    n2 = _rownorm(hv)
    hv = jnp.where(n2 > maxn, hv / n2 * maxn, hv)
    pn = _rownorm(hv)
    h = _artanh(pn) * hv / pn
    feats_o[...] = h
    pos = pos_ref[...]
    z = jnp.zeros((h.shape[0], TBL - EMB - 3), jnp.float32)
    tabr_o[...] = jnp.concatenate([_dotT(h, w1a[...]), pos, z], axis=-1)
    tabc_o[...] = jnp.concatenate([_dotT(h, w1b[...]), pos, z], axis=-1)


def _edge_body(gsum_ref, s_ref, wefft, beff, w2, b2, m2_o):
    m1 = gsum_ref[...] + _dotT(s_ref[...], wefft[...]) + beff[...]
    m = _silu(m1)
    m2_o[...] = _silu(_dotT(m, w2[...]) + b2[...])


def _node_body(feats_ref, agga_ref, aggb_ref, coors_ref, batch_ref, temb,
               wn0a, wn0b, bn0, wn1, bn1, wl1, bl1, lng, lnb, wl2, bl2,
               *rest, final):
    feats = feats_ref[...]
    agg = agga_ref[...] + aggb_ref[...]
    hm = _silu(_dotT(feats, wn0a[...]) + _dotT(agg, wn0b[...]) + bn0[...])
    h2 = _dotT(hm, wn1[...]) + bn1[...]
    f_out = feats + h2
    coors = coors_ref[...]
    nsq = (jnp.sum(coors * coors, axis=-1, keepdims=True)
           + jnp.sum(f_out * f_out, axis=-1, keepdims=True))
    un = jnp.clip(jnp.sqrt(nsq), 1e-15)
    fac = jnp.tanh(un) / un
    corr = coors * fac
    f2 = f_out * fac
    iot = lax.broadcasted_iota(jnp.int32, (f2.shape[0], G), 1)
    oh = (batch_ref[...] == iot).astype(jnp.float32)
    st = jnp.dot(oh, temb[...], preferred_element_type=jnp.float32,
                 precision=lax.Precision.HIGHEST)
    scale = st[:, :EMB]
    shift = st[:, EMB:]
    f3 = f2 * (scale + 1.0) + shift
    f = _silu(_dotT(f3, wl1[...]) + bl1[...])
    mu = jnp.mean(f, axis=-1, keepdims=True)
    var = jnp.mean((f - mu) * (f - mu), axis=-1, keepdims=True)
    fn = (f - mu) / jnp.sqrt(var + 1e-5) * lng[...] + lnb[...]
    f4 = _dotT(fn, wl2[...]) + bl2[...]
    if final:
        wlin, blin, out_o = rest
        out_o[...] = _dotT(f4, wlin[...]) + blin[...]
    else:
        w1a, w1b, feats_o, corr_o, tabr_o, tabc_o = rest
        feats_o[...] = f4
        corr_o[...] = corr
        z = jnp.zeros((f4.shape[0], TBL - EMB - 3), jnp.float32)
        tabr_o[...] = jnp.concatenate([_dotT(f4, w1a[...]), corr, z], axis=-1)
        tabc_o[...] = jnp.concatenate([_dotT(f4, w1b[...]), corr, z], axis=-1)


def _full(a):
    return pl.BlockSpec(a.shape, lambda i: (0,) * a.ndim)


def _blk(shape):
    return pl.BlockSpec(shape, lambda i: (i,) + (0,) * (len(shape) - 1))


def _pad_rows(a, n):
    return jnp.pad(a, ((0, n - a.shape[0]),) + ((0, 0),) * (a.ndim - 1))


NC = 2
NS = 16
NW = NC * NS
EPW = E_PAD // NW
EB = 128
NBLK = EPW // EB
NPT = N_PAD // NS


def _sc_mesh():
    return plsc.VectorSubcoreMesh(core_axis_name="c", subcore_axis_name="s")


def _gather_gsum(tabr, tabc, row, col, w1c):
    @functools.partial(
        pl.kernel,
        out_type=jax.ShapeDtypeStruct((E_PAD, EMB), jnp.float32),
        mesh=_sc_mesh(),
        scratch_types=[
            pltpu.VMEM((EB,), jnp.int32),
            pltpu.VMEM((EB,), jnp.int32),
            pltpu.VMEM((EB, TBL), jnp.float32),
            pltpu.VMEM((EB, TBL), jnp.float32),
            pltpu.VMEM((EB, EMB), jnp.float32),
            pltpu.VMEM((EMB,), jnp.float32),
            pltpu.SemaphoreType.DMA,
            pltpu.SemaphoreType.DMA,
        ],
    )
    def gk(tabr_h, tabc_h, row_h, col_h, w1c_h, out_h,
           idxr, idxc, bufr, bufc, outb, w1cv, sem1, sem2):
        wid = lax.axis_index("s") * NC + lax.axis_index("c")
        base = wid * EPW
        pltpu.sync_copy(w1c_h, w1cv)

        def blk(i, carry):
            off = base + i * EB
            pltpu.sync_copy(row_h.at[pl.ds(off, EB)], idxr)
            pltpu.sync_copy(col_h.at[pl.ds(off, EB)], idxc)
            cr = pltpu.async_copy(tabr_h.at[idxr], bufr, sem1)
            cc = pltpu.async_copy(tabc_h.at[idxc], bufc, sem2)
            cr.wait()
            cc.wait()

            def edge(e, c2):
                d = bufr[e, pl.ds(EMB, 16)] - bufc[e, pl.ds(EMB, 16)]
                sq = d * d
                rd = sq[0] + sq[1] + sq[2]
                for j in range(8):
                    sl = pl.ds(j * 16, 16)
                    outb[e, sl] = bufr[e, sl] + bufc[e, sl] + rd * w1cv[sl]
                return c2

            lax.fori_loop(0, EB, edge, 0)
            pltpu.sync_copy(outb, out_h.at[pl.ds(off, EB)])
            return carry

        lax.fori_loop(0, NBLK, blk, 0)

    return gk(tabr, tabc, row, col, w1c)


def _segsum(m2, row):
    @functools.partial(
        pl.kernel,
        out_type=jax.ShapeDtypeStruct((NC, N_PAD, EMB), jnp.float32),
        mesh=_sc_mesh(),
        scratch_types=[
            pltpu.VMEM((EB,), jnp.int32),
            pltpu.VMEM((EB, EMB), jnp.float32),
            pltpu.VMEM_SHARED((N_PAD, EMB), jnp.float32),
        ],
    )
    def sk(m2_h, row_h, out_h, idx, buf, acc):
        c = lax.axis_index("c")
        s = lax.axis_index("s")
        wid = s * NC + c
        base = wid * EPW

        def zr(r, carry):
            for j in range(8):
                buf[r, pl.ds(j * 16, 16)] = jnp.zeros((16,), jnp.float32)
            return carry

        lax.fori_loop(0, EB, zr, 0)
        for k in range(NPT // EB):
            pltpu.sync_copy(buf, acc.at[pl.ds(s * NPT + k * EB, EB)])
        plsc.subcore_barrier()

        def blk(i, carry):
            off = base + i * EB
            pltpu.sync_copy(row_h.at[pl.ds(off, EB)], idx)
            pltpu.sync_copy(m2_h.at[pl.ds(off, EB)], buf)
            pltpu.sync_copy(buf, acc.at[idx], add=True)
            return carry

        lax.fori_loop(0, NBLK, blk, 0)
        plsc.subcore_barrier()
        pltpu.sync_copy(acc.at[pl.ds(s * NPT, NPT)],
                        out_h.at[c, pl.ds(s * NPT, NPT)])

    out = sk(m2, row)
    return out[0], out[1]


def kernel(x, pos, extra_x, edge_index, edge_attr, batch, time, params):
    f32 = jnp.float32
    lyr = params["layers"]
    w1 = [l["edge_mlp"][0]["w"] for l in lyr]
    w1a = [w[:, :EMB] for w in w1]
    w1b = [w[:, EMB:2 * EMB] for w in w1]
    w1c = [w[:, 2 * EMB] for w in w1]
    w1d = [w[:, 2 * EMB + 1:] for w in w1]
    b1 = [l["edge_mlp"][0]["b"].reshape(1, EMB) for l in lyr]
    w2 = [l["edge_mlp"][1]["w"] for l in lyr]
    b2 = [l["edge_mlp"][1]["b"].reshape(1, EMB) for l in lyr]
    wn0 = [l["node_mlp"][0]["w"] for l in lyr]
    bn0 = [l["node_mlp"][0]["b"].reshape(1, EMB) for l in lyr]
    wn1 = [l["node_mlp"][1]["w"] for l in lyr]
    bn1 = [l["node_mlp"][1]["b"].reshape(1, EMB) for l in lyr]
    wl1 = [l["ff"]["l1"]["w"] for l in lyr]
    bl1 = [l["ff"]["l1"]["b"].reshape(1, EMB) for l in lyr]
    lng = [l["ff"]["ln_g"].reshape(1, EMB) for l in lyr]
    lnb = [l["ff"]["ln_b"].reshape(1, EMB) for l in lyr]
    wl2 = [l["ff"]["l2"]["w"] for l in lyr]
    bl2 = [l["ff"]["l2"]["b"].reshape(1, EMB) for l in lyr]

    prep_out = pl.pallas_call(
        _prep_body,
        out_shape=[
            jax.ShapeDtypeStruct((G, 2 * EMB), f32),
            jax.ShapeDtypeStruct((G, 2 * EMB), f32),
            jax.ShapeDtypeStruct((EMB, 4), f32),
            jax.ShapeDtypeStruct((1, EMB), f32),
            jax.ShapeDtypeStruct((EMB, EMB), f32),
            jax.ShapeDtypeStruct((1, EMB), f32),
        ],
    )(time,
      params["time_mlp"][0]["w"], params["time_mlp"][0]["b"].reshape(1, -1),
      params["time_mlp"][1]["w"], params["time_mlp"][1]["b"].reshape(1, -1),
      lyr[0]["time_proj"]["w"], lyr[0]["time_proj"]["b"].reshape(1, -1),
      lyr[1]["time_proj"]["w"], lyr[1]["time_proj"]["b"].reshape(1, -1),
      params["edge_enc"]["w"], params["edge_enc"]["b"].reshape(1, -1),
      w1d[0], b1[0],
      lyr[0]["edge_upd"]["w"], lyr[0]["edge_upd"]["b"].reshape(1, -1),
      w1d[1], b1[1])
    temb0, temb1, wefft0, beff0, wefft1, beff1 = prep_out

    x_p = _pad_rows(x, N_PAD)
    ex_p = _pad_rows(extra_x, N_PAD)
    pos_p = _pad_rows(pos, N_PAD)
    batch_p = _pad_rows(batch.reshape(N, 1), N_PAD)
    npad = E_PAD - E
    fill = N + (jnp.arange(npad, dtype=jnp.int32) % (N_PAD - N))
    row_p = jnp.concatenate([edge_index[0], fill])
    col_p = jnp.concatenate([edge_index[1], fill])
    eattr_p = _pad_rows(edge_attr, E_PAD)

    grid_n = (N_PAD // BN,)
    feats0, tabr0, tabc0 = pl.pallas_call(
        _pre_body,
        grid=grid_n,
        in_specs=[_blk((BN, 12)), _blk((BN, 3)), _blk((BN, 3)),
                  _full(params["hyp_w"]), _full(w1a[0]), _full(w1b[0])],
        out_specs=[_blk((BN, EMB)), _blk((BN, TBL)), _blk((BN, TBL))],
        out_shape=[
            jax.ShapeDtypeStruct((N_PAD, EMB), f32),
            jax.ShapeDtypeStruct((N_PAD, TBL), f32),
            jax.ShapeDtypeStruct((N_PAD, TBL), f32),
        ],
    )(x_p, ex_p, pos_p, params["hyp_w"], w1a[0], w1b[0])

    grid_e = (E_PAD // BE,)

    def edge_mlp(gsum, s_e, wefft, beff, w2_, b2_):
        ds = s_e.shape[1]
        return pl.pallas_call(
            _edge_body,
            grid=grid_e,
            in_specs=[_blk((BE, EMB)), _blk((BE, ds)), _full(wefft),
                      _full(beff), _full(w2_), _full(b2_)],
            out_specs=_blk((BE, EMB)),
            out_shape=jax.ShapeDtypeStruct((E_PAD, EMB), f32),
        )(gsum, s_e, wefft, beff, w2_, b2_)

    def node_update(li, feats, agga, aggb, coors, temb, final):
        body = functools.partial(_node_body, final=final)
        common = [feats, agga, aggb, coors, batch_p, temb,
                  wn0[li][:, :EMB], wn0[li][:, EMB:], bn0[li], wn1[li],
                  bn1[li], wl1[li], bl1[li], lng[li], lnb[li], wl2[li],
                  bl2[li]]
        common_specs = [_blk((BN, EMB)), _blk((BN, EMB)), _blk((BN, EMB)),
                        _blk((BN, 3)), _blk((BN, 1)), _full(temb)] + \
            [_full(a) for a in common[6:]]
        if final:
            wlin = params["lin"]["w"]
            blin = params["lin"]["b"].reshape(1, -1)
            return pl.pallas_call(
                body, grid=grid_n,
                in_specs=common_specs + [_full(wlin), _full(blin)],
                out_specs=_blk((BN, 4)),
                out_shape=jax.ShapeDtypeStruct((N_PAD, 4), f32),
            )(*common, wlin, blin)
        return pl.pallas_call(
            body, grid=grid_n,
            in_specs=common_specs + [_full(w1a[li + 1]), _full(w1b[li + 1])],
            out_specs=[_blk((BN, EMB)), _blk((BN, 3)), _blk((BN, TBL)),
                       _blk((BN, TBL))],
            out_shape=[
                jax.ShapeDtypeStruct((N_PAD, EMB), f32),
                jax.ShapeDtypeStruct((N_PAD, 3), f32),
                jax.ShapeDtypeStruct((N_PAD, TBL), f32),
                jax.ShapeDtypeStruct((N_PAD, TBL), f32),
            ])(*common, w1a[li + 1], w1b[li + 1])

    gsum0 = _gather_gsum(tabr0, tabc0, row_p, col_p, w1c[0])
    m2_0 = edge_mlp(gsum0, eattr_p, wefft0, beff0, w2[0], b2[0])
    agg0a, agg0b = _segsum(m2_0, row_p)
    feats1, corr1, tabr1, tabc1 = node_update(
        0, feats0, agg0a, agg0b, pos_p, temb0, final=False)

    gsum1 = _gather_gsum(tabr1, tabc1, row_p, col_p, w1c[1])
    m2_1 = edge_mlp(gsum1, m2_0, wefft1, beff1, w2[1], b2[1])
    agg1a, agg1b = _segsum(m2_1, row_p)
    out_p = node_update(1, feats1, agg1a, agg1b, corr1, temb1, final=True)

    return out_p[:N]


# speedup vs baseline: 2.6235x; 1.2419x over previous
import functools

import jax
import jax.numpy as jnp
from jax import lax
from jax.experimental import pallas as pl
from jax.experimental.pallas import tpu as pltpu
from jax.experimental.pallas import tpu_sc as plsc

N = 10000
N_PAD = 10240
E = 320000
E_PAD = 323584
G = 8
EMB = 128
TBL = 256
BN = 1024
BE = 1024


def _dotT(x, w):
    return lax.dot_general(x, w, (((1,), (1,)), ((), ())),
                           preferred_element_type=jnp.float32,
                           precision=lax.Precision.HIGHEST)


def _silu(x):
    return x * jax.nn.sigmoid(x)


def _artanh(x):
    x = jnp.clip(x, -1.0 + 1e-7, 1.0 - 1e-7)
    return 0.5 * (jnp.log1p(x) - jnp.log1p(-x))


def _rownorm(x):
    return jnp.clip(jnp.sqrt(jnp.sum(x * x, axis=-1, keepdims=True)), 1e-15)


def _prep_body(time_ref, tm0w, tm0b, tm1w, tm1b, tp0w, tp0b, tp1w, tp1b,
               wenc, benc, w1d0, b10, wu0, bu0, w1d1, b11,
               temb0_o, temb1_o, weffT0_o, beff0_o, weffT1_o, beff1_o):
    t1 = _silu(_dotT(time_ref[...], tm0w[...]) + tm0b[...])
    t = _dotT(t1, tm1w[...]) + tm1b[...]
    st = _silu(t)
    temb0_o[...] = _dotT(st, tp0w[...]) + tp0b[...]
    temb1_o[...] = _dotT(st, tp1w[...]) + tp1b[...]
    weffT0_o[...] = jnp.dot(w1d0[...], wenc[...],
                            preferred_element_type=jnp.float32,
                            precision=lax.Precision.HIGHEST)
    beff0_o[...] = _dotT(benc[...], w1d0[...]) + b10[...]
    weffT1_o[...] = jnp.dot(w1d1[...], wu0[...],
                            preferred_element_type=jnp.float32,
                            precision=lax.Precision.HIGHEST)
    beff1_o[...] = _dotT(bu0[...], w1d1[...]) + b11[...]


def _pre_body(x_ref, ex_ref, pos_ref, hypw, w1a, w1b,
              feats_o, tabr_o, tabc_o):
    xc = jnp.concatenate([x_ref[...], ex_ref[...]], axis=-1)
    un = _rownorm(xc)
    u1 = jnp.tanh(un) * xc / un
    n = _rownorm(u1)
    maxn = 1.0 - 4e-3
    xh = jnp.where(n > maxn, u1 / n * maxn, u1)
    xn = _rownorm(xh)
    mx = _dotT(xh, hypw[...])
    mxn = _rownorm(mx)
    hv = jnp.tanh(mxn / xn * _artanh(xn)) * mx / mxn
    n2 = _rownorm(hv)
    hv = jnp.where(n2 > maxn, hv / n2 * maxn, hv)
    pn = _rownorm(hv)
    h = _artanh(pn) * hv / pn
    feats_o[...] = h
    pos = pos_ref[...]
    z = jnp.zeros((h.shape[0], TBL - EMB - 3), jnp.float32)
    tabr_o[...] = jnp.concatenate([_dotT(h, w1a[...]), pos, z], axis=-1)
    tabc_o[...] = jnp.concatenate([_dotT(h, w1b[...]), pos, z], axis=-1)


def _edge_body(gsum_ref, s_ref, wefft, beff, w2, b2, m2_o):
    m1 = gsum_ref[...] + _dotT(s_ref[...], wefft[...]) + beff[...]
    m = _silu(m1)
    m2_o[...] = _silu(_dotT(m, w2[...]) + b2[...])


def _node_body(feats_ref, agga_ref, aggb_ref, coors_ref, batch_ref, temb,
               wn0a, wn0b, bn0, wn1, bn1, wl1, bl1, lng, lnb, wl2, bl2,
               *rest, final):
    feats = feats_ref[...]
    agg = agga_ref[...] + aggb_ref[...]
    hm = _silu(_dotT(feats, wn0a[...]) + _dotT(agg, wn0b[...]) + bn0[...])
    h2 = _dotT(hm, wn1[...]) + bn1[...]
    f_out = feats + h2
    coors = coors_ref[...]
    nsq = (jnp.sum(coors * coors, axis=-1, keepdims=True)
           + jnp.sum(f_out * f_out, axis=-1, keepdims=True))
    un = jnp.clip(jnp.sqrt(nsq), 1e-15)
    fac = jnp.tanh(un) / un
    corr = coors * fac
    f2 = f_out * fac
    iot = lax.broadcasted_iota(jnp.int32, (f2.shape[0], G), 1)
    oh = (batch_ref[...] == iot).astype(jnp.float32)
    st = jnp.dot(oh, temb[...], preferred_element_type=jnp.float32,
                 precision=lax.Precision.HIGHEST)
    scale = st[:, :EMB]
    shift = st[:, EMB:]
    f3 = f2 * (scale + 1.0) + shift
    f = _silu(_dotT(f3, wl1[...]) + bl1[...])
    mu = jnp.mean(f, axis=-1, keepdims=True)
    var = jnp.mean((f - mu) * (f - mu), axis=-1, keepdims=True)
    fn = (f - mu) / jnp.sqrt(var + 1e-5) * lng[...] + lnb[...]
    f4 = _dotT(fn, wl2[...]) + bl2[...]
    if final:
        wlin, blin, out_o = rest
        out_o[...] = _dotT(f4, wlin[...]) + blin[...]
    else:
        w1a, w1b, feats_o, corr_o, tabr_o, tabc_o = rest
        feats_o[...] = f4
        corr_o[...] = corr
        z = jnp.zeros((f4.shape[0], TBL - EMB - 3), jnp.float32)
        tabr_o[...] = jnp.concatenate([_dotT(f4, w1a[...]), corr, z], axis=-1)
        tabc_o[...] = jnp.concatenate([_dotT(f4, w1b[...]), corr, z], axis=-1)


def _full(a):
    return pl.BlockSpec(a.shape, lambda i: (0,) * a.ndim)


def _blk(shape):
    return pl.BlockSpec(shape, lambda i: (i,) + (0,) * (len(shape) - 1))


def _pad_rows(a, n):
    return jnp.pad(a, ((0, n - a.shape[0]),) + ((0, 0),) * (a.ndim - 1))


NC = 2
NS = 16
NW = NC * NS
EPW = E_PAD // NW
EB = 128
NBLK = EPW // EB
GB = 64
GNB = EPW // GB
NPT = N_PAD // NS


def _sc_mesh():
    return plsc.VectorSubcoreMesh(core_axis_name="c", subcore_axis_name="s")


def _gather_gsum(tabr, tabc, row, col, w1c):
    @functools.partial(
        pl.kernel,
        out_type=jax.ShapeDtypeStruct((E_PAD, EMB), jnp.float32),
        mesh=_sc_mesh(),
        scratch_types=[
            pltpu.VMEM((EPW,), jnp.int32),
            pltpu.VMEM((EPW,), jnp.int32),
            pltpu.VMEM((GB, TBL), jnp.float32),
            pltpu.VMEM((GB, TBL), jnp.float32),
            pltpu.VMEM((GB, TBL), jnp.float32),
            pltpu.VMEM((GB, TBL), jnp.float32),
            pltpu.VMEM((GB, EMB), jnp.float32),
            pltpu.VMEM((GB, EMB), jnp.float32),
            pltpu.VMEM((EMB,), jnp.float32),
            pltpu.SemaphoreType.DMA,
            pltpu.SemaphoreType.DMA,
            pltpu.SemaphoreType.DMA,
            pltpu.SemaphoreType.DMA,
            pltpu.SemaphoreType.DMA,
            pltpu.SemaphoreType.DMA,
        ],
    )
    def gk(tabr_h, tabc_h, row_h, col_h, w1c_h, out_h,
           idxr_all, idxc_all, bufr0, bufc0, bufr1, bufc1, outb0, outb1,
           w1cv, semr0, semc0, semr1, semc1, semo0, semo1):
        wid = lax.axis_index("s") * NC + lax.axis_index("c")
        base = wid * EPW
        pltpu.sync_copy(w1c_h, w1cv)
        pltpu.sync_copy(row_h.at[pl.ds(base, EPW)], idxr_all)
        pltpu.sync_copy(col_h.at[pl.ds(base, EPW)], idxc_all)
        bufs = ((bufr0, bufc0, outb0, semr0, semc0, semo0),
                (bufr1, bufc1, outb1, semr1, semc1, semo1))

        def issue(g, s):
            br, bc, _, sr, sc_, _ = bufs[s]
            pltpu.async_copy(tabr_h.at[idxr_all.at[pl.ds(g * GB, GB)]], br, sr)
            pltpu.async_copy(tabc_h.at[idxc_all.at[pl.ds(g * GB, GB)]], bc, sc_)

        def wait_in(s):
            br, bc, _, sr, sc_, _ = bufs[s]
            pltpu.make_async_copy(tabr_h.at[idxr_all.at[pl.ds(0, GB)]],
                                  br, sr).wait()
            pltpu.make_async_copy(tabc_h.at[idxc_all.at[pl.ds(0, GB)]],
                                  bc, sc_).wait()

        def compute(s):
            br, bc, ob, *_ = bufs[s]

            def edge(e, c2):
                d = br[e, pl.ds(EMB, 16)] - bc[e, pl.ds(EMB, 16)]
                sq = d * d
                rd = sq[0] + sq[1] + sq[2]
                for j in range(8):
                    sl = pl.ds(j * 16, 16)
                    ob[e, sl] = br[e, sl] + bc[e, sl] + rd * w1cv[sl]
                return c2

            lax.fori_loop(0, GB, edge, 0, unroll=2)

        def write(g, s):
            ob, so = bufs[s][2], bufs[s][5]
            pltpu.async_copy(ob, out_h.at[pl.ds(base + g * GB, GB)], so)

        def wait_out(s):
            ob, so = bufs[s][2], bufs[s][5]
            pltpu.make_async_copy(ob, out_h.at[pl.ds(0, GB)], so).wait()

        issue(0, 0)
        issue(1, 1)

        def body(it, carry):
            g = it * 2

            def phase(s, gg):
                wait_in(s)

                @pl.when(it > 0)
                def _():
                    wait_out(s)

                compute(s)
                write(gg, s)

                @pl.when(gg + 2 < GNB)
                def _():
                    issue(gg + 2, s)

            phase(0, g)
            phase(1, g + 1)
            return carry

        lax.fori_loop(0, GNB // 2, body, 0)
        wait_out(0)
        wait_out(1)

    return gk(tabr, tabc, row, col, w1c)


def _segsum(m2, row):
    @functools.partial(
        pl.kernel,
        out_type=jax.ShapeDtypeStruct((NC, N_PAD, EMB), jnp.float32),
        mesh=_sc_mesh(),
        scratch_types=[
            pltpu.VMEM((EB,), jnp.int32),
            pltpu.VMEM((EB, EMB), jnp.float32),
            pltpu.VMEM_SHARED((N_PAD, EMB), jnp.float32),
        ],
    )
    def sk(m2_h, row_h, out_h, idx, buf, acc):
        c = lax.axis_index("c")
        s = lax.axis_index("s")
        wid = s * NC + c
        base = wid * EPW

        def zr(r, carry):
            for j in range(8):
                buf[r, pl.ds(j * 16, 16)] = jnp.zeros((16,), jnp.float32)
            return carry

        lax.fori_loop(0, EB, zr, 0)
        for k in range(NPT // EB):
            pltpu.sync_copy(buf, acc.at[pl.ds(s * NPT + k * EB, EB)])
        plsc.subcore_barrier()

        def blk(i, carry):
            off = base + i * EB
            pltpu.sync_copy(row_h.at[pl.ds(off, EB)], idx)
            pltpu.sync_copy(m2_h.at[pl.ds(off, EB)], buf)
            pltpu.sync_copy(buf, acc.at[idx], add=True)
            return carry

        lax.fori_loop(0, NBLK, blk, 0)
        plsc.subcore_barrier()
        pltpu.sync_copy(acc.at[pl.ds(s * NPT, NPT)],
                        out_h.at[c, pl.ds(s * NPT, NPT)])

    out = sk(m2, row)
    return out[0], out[1]


def kernel(x, pos, extra_x, edge_index, edge_attr, batch, time, params):
    f32 = jnp.float32
    lyr = params["layers"]
    w1 = [l["edge_mlp"][0]["w"] for l in lyr]
    w1a = [w[:, :EMB] for w in w1]
    w1b = [w[:, EMB:2 * EMB] for w in w1]
    w1c = [w[:, 2 * EMB] for w in w1]
    w1d = [w[:, 2 * EMB + 1:] for w in w1]
    b1 = [l["edge_mlp"][0]["b"].reshape(1, EMB) for l in lyr]
    w2 = [l["edge_mlp"][1]["w"] for l in lyr]
    b2 = [l["edge_mlp"][1]["b"].reshape(1, EMB) for l in lyr]
    wn0 = [l["node_mlp"][0]["w"] for l in lyr]
    bn0 = [l["node_mlp"][0]["b"].reshape(1, EMB) for l in lyr]
    wn1 = [l["node_mlp"][1]["w"] for l in lyr]
    bn1 = [l["node_mlp"][1]["b"].reshape(1, EMB) for l in lyr]
    wl1 = [l["ff"]["l1"]["w"] for l in lyr]
    bl1 = [l["ff"]["l1"]["b"].reshape(1, EMB) for l in lyr]
    lng = [l["ff"]["ln_g"].reshape(1, EMB) for l in lyr]
    lnb = [l["ff"]["ln_b"].reshape(1, EMB) for l in lyr]
    wl2 = [l["ff"]["l2"]["w"] for l in lyr]
    bl2 = [l["ff"]["l2"]["b"].reshape(1, EMB) for l in lyr]

    prep_out = pl.pallas_call(
        _prep_body,
        out_shape=[
            jax.ShapeDtypeStruct((G, 2 * EMB), f32),
            jax.ShapeDtypeStruct((G, 2 * EMB), f32),
            jax.ShapeDtypeStruct((EMB, 4), f32),
            jax.ShapeDtypeStruct((1, EMB), f32),
            jax.ShapeDtypeStruct((EMB, EMB), f32),
            jax.ShapeDtypeStruct((1, EMB), f32),
        ],
    )(time,
      params["time_mlp"][0]["w"], params["time_mlp"][0]["b"].reshape(1, -1),
      params["time_mlp"][1]["w"], params["time_mlp"][1]["b"].reshape(1, -1),
      lyr[0]["time_proj"]["w"], lyr[0]["time_proj"]["b"].reshape(1, -1),
      lyr[1]["time_proj"]["w"], lyr[1]["time_proj"]["b"].reshape(1, -1),
      params["edge_enc"]["w"], params["edge_enc"]["b"].reshape(1, -1),
      w1d[0], b1[0],
      lyr[0]["edge_upd"]["w"], lyr[0]["edge_upd"]["b"].reshape(1, -1),
      w1d[1], b1[1])
    temb0, temb1, wefft0, beff0, wefft1, beff1 = prep_out

    x_p = _pad_rows(x, N_PAD)
    ex_p = _pad_rows(extra_x, N_PAD)
    pos_p = _pad_rows(pos, N_PAD)
    batch_p = _pad_rows(batch.reshape(N, 1), N_PAD)
    npad = E_PAD - E
    fill = N + (jnp.arange(npad, dtype=jnp.int32) % (N_PAD - N))
    row_p = jnp.concatenate([edge_index[0], fill])
    col_p = jnp.concatenate([edge_index[1], fill])
    eattr_p = _pad_rows(edge_attr, E_PAD)

    grid_n = (N_PAD // BN,)
    feats0, tabr0, tabc0 = pl.pallas_call(
        _pre_body,
        grid=grid_n,
        in_specs=[_blk((BN, 12)), _blk((BN, 3)), _blk((BN, 3)),
                  _full(params["hyp_w"]), _full(w1a[0]), _full(w1b[0])],
        out_specs=[_blk((BN, EMB)), _blk((BN, TBL)), _blk((BN, TBL))],
        out_shape=[
            jax.ShapeDtypeStruct((N_PAD, EMB), f32),
            jax.ShapeDtypeStruct((N_PAD, TBL), f32),
            jax.ShapeDtypeStruct((N_PAD, TBL), f32),
        ],
    )(x_p, ex_p, pos_p, params["hyp_w"], w1a[0], w1b[0])

    grid_e = (E_PAD // BE,)

    def edge_mlp(gsum, s_e, wefft, beff, w2_, b2_):
        ds = s_e.shape[1]
        return pl.pallas_call(
            _edge_body,
            grid=grid_e,
            in_specs=[_blk((BE, EMB)), _blk((BE, ds)), _full(wefft),
                      _full(beff), _full(w2_), _full(b2_)],
            out_specs=_blk((BE, EMB)),
            out_shape=jax.ShapeDtypeStruct((E_PAD, EMB), f32),
        )(gsum, s_e, wefft, beff, w2_, b2_)

    def node_update(li, feats, agga, aggb, coors, temb, final):
        body = functools.partial(_node_body, final=final)
        common = [feats, agga, aggb, coors, batch_p, temb,
                  wn0[li][:, :EMB], wn0[li][:, EMB:], bn0[li], wn1[li],
                  bn1[li], wl1[li], bl1[li], lng[li], lnb[li], wl2[li],
                  bl2[li]]
        common_specs = [_blk((BN, EMB)), _blk((BN, EMB)), _blk((BN, EMB)),
                        _blk((BN, 3)), _blk((BN, 1)), _full(temb)] + \
            [_full(a) for a in common[6:]]
        if final:
            wlin = params["lin"]["w"]
            blin = params["lin"]["b"].reshape(1, -1)
            return pl.pallas_call(
                body, grid=grid_n,
                in_specs=common_specs + [_full(wlin), _full(blin)],
                out_specs=_blk((BN, 4)),
                out_shape=jax.ShapeDtypeStruct((N_PAD, 4), f32),
            )(*common, wlin, blin)
        return pl.pallas_call(
            body, grid=grid_n,
            in_specs=common_specs + [_full(w1a[li + 1]), _full(w1b[li + 1])],
            out_specs=[_blk((BN, EMB)), _blk((BN, 3)), _blk((BN, TBL)),
                       _blk((BN, TBL))],
            out_shape=[
                jax.ShapeDtypeStruct((N_PAD, EMB), f32),
                jax.ShapeDtypeStruct((N_PAD, 3), f32),
                jax.ShapeDtypeStruct((N_PAD, TBL), f32),
                jax.ShapeDtypeStruct((N_PAD, TBL), f32),
            ])(*common, w1a[li + 1], w1b[li + 1])

    gsum0 = _gather_gsum(tabr0, tabc0, row_p, col_p, w1c[0])
    m2_0 = edge_mlp(gsum0, eattr_p, wefft0, beff0, w2[0], b2[0])
    agg0a, agg0b = _segsum(m2_0, row_p)
    feats1, corr1, tabr1, tabc1 = node_update(
        0, feats0, agg0a, agg0b, pos_p, temb0, final=False)

    gsum1 = _gather_gsum(tabr1, tabc1, row_p, col_p, w1c[1])
    m2_1 = edge_mlp(gsum1, m2_0, wefft1, beff1, w2[1], b2[1])
    agg1a, agg1b = _segsum(m2_1, row_p)
    out_p = node_update(1, feats1, agg1a, agg1b, corr1, temb1, final=True)

    return out_p[:N]


# speedup vs baseline: 3.2651x; 1.2445x over previous
import functools

import jax
import jax.numpy as jnp
from jax import lax
from jax.experimental import pallas as pl
from jax.experimental.pallas import tpu as pltpu
from jax.experimental.pallas import tpu_sc as plsc

N = 10000
N_PAD = 10240
E = 320000
E_PAD = 323584
G = 8
EMB = 128
TBL = 256
BN = 1024
BE = 1024


def _dotT(x, w):
    return lax.dot_general(x, w, (((1,), (1,)), ((), ())),
                           preferred_element_type=jnp.float32)


def _silu(x):
    return x * jax.nn.sigmoid(x)


def _artanh(x):
    x = jnp.clip(x, -1.0 + 1e-7, 1.0 - 1e-7)
    return 0.5 * (jnp.log1p(x) - jnp.log1p(-x))


def _rownorm(x):
    return jnp.clip(jnp.sqrt(jnp.sum(x * x, axis=-1, keepdims=True)), 1e-15)


def _prep_body(time_ref, tm0wT, tm0b, tm1w, tm1b, tp0w, tp0b, tp1w, tp1b,
               temb0_o, temb1_o):
    t1 = _silu(time_ref[...] * tm0wT[...] + tm0b[...])
    t = _dotT(t1, tm1w[...]) + tm1b[...]
    st = _silu(t)
    temb0_o[...] = _dotT(st, tp0w[...]) + tp0b[...]
    temb1_o[...] = _dotT(st, tp1w[...]) + tp1b[...]


def _pre_body(x_ref, ex_ref, pos_ref, hypw, feats_o, tab_o):
    xc = jnp.concatenate([x_ref[...], ex_ref[...]], axis=-1)
    un = _rownorm(xc)
    u1 = jnp.tanh(un) * xc / un
    n = _rownorm(u1)
    maxn = 1.0 - 4e-3
    xh = jnp.where(n > maxn, u1 / n * maxn, u1)
    xn = _rownorm(xh)
    mx = _dotT(xh, hypw[...])
    mxn = _rownorm(mx)
    hv = jnp.tanh(mxn / xn * _artanh(xn)) * mx / mxn
    n2 = _rownorm(hv)
    hv = jnp.where(n2 > maxn, hv / n2 * maxn, hv)
    pn = _rownorm(hv)
    h = _artanh(pn) * hv / pn
    feats_o[...] = h
    z = jnp.zeros((h.shape[0], TBL - EMB - 3), jnp.float32)
    tab_o[...] = jnp.concatenate([h, pos_ref[...], z], axis=-1)


def _edge_body(fr_ref, fc_ref, rl_ref, s_ref, we, be, w1, b1, w2, b2, m2_o):
    ea = _dotT(s_ref[...], we[...]) + be[...]
    e_in = jnp.concatenate([fr_ref[0], fc_ref[0], rl_ref[0][:, :1], ea],
                           axis=-1)
    m1 = _dotT(e_in, w1[...]) + b1[...]
    m = _silu(m1)
    m2_o[...] = _silu(_dotT(m, w2[...]) + b2[...])


def _node_body(feats_ref, agga_ref, aggb_ref, coors_ref, batch_ref, temb,
               wn0, bn0, wn1, bn1, wl1, bl1, lng, lnb, wl2, bl2,
               *rest, final):
    feats = feats_ref[...]
    agg = agga_ref[...] + aggb_ref[...]
    hin = jnp.concatenate([feats, agg], axis=-1)
    hm = _silu(_dotT(hin, wn0[...]) + bn0[...])
    h2 = _dotT(hm, wn1[...]) + bn1[...]
    f_out = feats + h2
    coors = coors_ref[...]
    nsq = (jnp.sum(coors * coors, axis=-1, keepdims=True)
           + jnp.sum(f_out * f_out, axis=-1, keepdims=True))
    un = jnp.clip(jnp.sqrt(nsq), 1e-15)
    fac = jnp.tanh(un) / un
    corr = coors * fac
    f2 = f_out * fac
    iot = lax.broadcasted_iota(jnp.int32, (f2.shape[0], G), 1)
    oh = (batch_ref[...] == iot).astype(jnp.float32)
    st = jnp.dot(oh, temb[...], preferred_element_type=jnp.float32,
                 precision=lax.Precision.HIGHEST)
    scale = st[:, :EMB]
    shift = st[:, EMB:]
    f3 = f2 * (scale + 1.0) + shift
    f = _silu(_dotT(f3, wl1[...]) + bl1[...])
    mu = jnp.mean(f, axis=-1, keepdims=True)
    var = jnp.mean((f - mu) * (f - mu), axis=-1, keepdims=True)
    fn = (f - mu) / jnp.sqrt(var + 1e-5) * lng[...] + lnb[...]
    f4 = _dotT(fn, wl2[...]) + bl2[...]
    if final:
        wlin, blin, out_o = rest
        out_o[...] = _dotT(f4, wlin[...]) + blin[...]
    else:
        feats_o, corr_o, tab_o = rest
        feats_o[...] = f4
        corr_o[...] = corr
        z = jnp.zeros((f4.shape[0], TBL - EMB - 3), jnp.float32)
        tab_o[...] = jnp.concatenate([f4, corr, z], axis=-1)


def _full(a):
    return pl.BlockSpec(a.shape, lambda i: (0,) * a.ndim)


def _blk(shape):
    return pl.BlockSpec(shape, lambda i: (i,) + (0,) * (len(shape) - 1))


def _pad_rows(a, n):
    return jnp.pad(a, ((0, n - a.shape[0]),) + ((0, 0),) * (a.ndim - 1))


NC = 2
NS = 16
NW = NC * NS
EPW = E_PAD // NW
EB = 128
NBLK = EPW // EB
GB = 32
GNB = EPW // GB
NPT = N_PAD // NS


def _sc_mesh():
    return plsc.VectorSubcoreMesh(core_axis_name="c", subcore_axis_name="s")


def _gather_pairs(tab, row, col):
    @functools.partial(
        pl.kernel,
        out_type=jax.ShapeDtypeStruct((3, E_PAD, EMB), jnp.float32),
        mesh=_sc_mesh(),
        scratch_types=[
            pltpu.VMEM((EPW,), jnp.int32),
            pltpu.VMEM((EPW,), jnp.int32),
            pltpu.VMEM((GB, TBL), jnp.float32),
            pltpu.VMEM((GB, TBL), jnp.float32),
            pltpu.VMEM((GB, TBL), jnp.float32),
            pltpu.VMEM((GB, TBL), jnp.float32),
            pltpu.VMEM((GB, EMB), jnp.float32),
            pltpu.VMEM((GB, EMB), jnp.float32),
            pltpu.VMEM((GB, EMB), jnp.float32),
            pltpu.VMEM((GB, EMB), jnp.float32),
            pltpu.VMEM((GB, EMB), jnp.float32),
            pltpu.VMEM((GB, EMB), jnp.float32),
            pltpu.SemaphoreType.DMA,
            pltpu.SemaphoreType.DMA,
            pltpu.SemaphoreType.DMA,
            pltpu.SemaphoreType.DMA,
            pltpu.SemaphoreType.DMA,
            pltpu.SemaphoreType.DMA,
        ],
    )
    def gk(tab_h, row_h, col_h, out_h,
           idxr_all, idxc_all, bufr0, bufc0, bufr1, bufc1,
           obr0, obc0, obl0, obr1, obc1, obl1,
           semr0, semc0, semr1, semc1, semo0, semo1):
        wid = lax.axis_index("s") * NC + lax.axis_index("c")
        base = wid * EPW
        pltpu.sync_copy(row_h.at[pl.ds(base, EPW)], idxr_all)
        pltpu.sync_copy(col_h.at[pl.ds(base, EPW)], idxc_all)
        bufs = ((bufr0, bufc0, (obr0, obc0, obl0), semr0, semc0, semo0),
                (bufr1, bufc1, (obr1, obc1, obl1), semr1, semc1, semo1))

        def issue(g, s):
            br, bc, _, sr, sc_, _ = bufs[s]
            pltpu.async_copy(tab_h.at[idxr_all.at[pl.ds(g * GB, GB)]], br, sr)
            pltpu.async_copy(tab_h.at[idxc_all.at[pl.ds(g * GB, GB)]], bc, sc_)

        def wait_in(s):
            br, bc, _, sr, sc_, _ = bufs[s]
            pltpu.make_async_copy(tab_h.at[idxr_all.at[pl.ds(0, GB)]],
                                  br, sr).wait()
            pltpu.make_async_copy(tab_h.at[idxc_all.at[pl.ds(0, GB)]],
                                  bc, sc_).wait()

        def compute(s):
            br, bc, (obr, obc, obl), *_ = bufs[s]

            def edge(e, c2):
                d = br[e, pl.ds(EMB, 16)] - bc[e, pl.ds(EMB, 16)]
                sq = d * d
                rd = sq[0] + sq[1] + sq[2]
                for j in range(8):
                    obr[e, pl.ds(j * 16, 16)] = br[e, pl.ds(j * 16, 16)]
                    obc[e, pl.ds(j * 16, 16)] = bc[e, pl.ds(j * 16, 16)]
                obl[e, pl.ds(0, 16)] = jnp.broadcast_to(rd, (16,))
                return c2

            lax.fori_loop(0, GB, edge, 0, unroll=2)

        def write(g, s):
            (obr, obc, obl), so = bufs[s][2], bufs[s][5]
            pltpu.async_copy(obr, out_h.at[0, pl.ds(base + g * GB, GB)], so)
            pltpu.async_copy(obc, out_h.at[1, pl.ds(base + g * GB, GB)], so)
            pltpu.async_copy(obl, out_h.at[2, pl.ds(base + g * GB, GB)], so)

        def wait_out(s):
            (obr, obc, obl), so = bufs[s][2], bufs[s][5]
            pltpu.make_async_copy(obr, out_h.at[0, pl.ds(0, GB)], so).wait()
            pltpu.make_async_copy(obc, out_h.at[1, pl.ds(0, GB)], so).wait()
            pltpu.make_async_copy(obl, out_h.at[2, pl.ds(0, GB)], so).wait()

        issue(0, 0)
        issue(1, 1)

        def body(it, carry):
            g = it * 2

            def phase(s, gg):
                wait_in(s)

                @pl.when(it > 0)
                def _():
                    wait_out(s)

                compute(s)
                write(gg, s)

                @pl.when(gg + 2 < GNB)
                def _():
                    issue(gg + 2, s)

            phase(0, g)
            phase(1, g + 1)
            return carry

        lax.fori_loop(0, GNB // 2, body, 0)
        wait_out(0)
        wait_out(1)

    return gk(tab, row, col)


def _segsum(m2, row):
    @functools.partial(
        pl.kernel,
        out_type=jax.ShapeDtypeStruct((NC, N_PAD, EMB), jnp.float32),
        mesh=_sc_mesh(),
        scratch_types=[
            pltpu.VMEM((EB,), jnp.int32),
            pltpu.VMEM((EB, EMB), jnp.float32),
            pltpu.VMEM_SHARED((N_PAD, EMB), jnp.float32),
        ],
    )
    def sk(m2_h, row_h, out_h, idx, buf, acc):
        c = lax.axis_index("c")
        s = lax.axis_index("s")
        wid = s * NC + c
        base = wid * EPW

        def zr(r, carry):
            for j in range(8):
                buf[r, pl.ds(j * 16, 16)] = jnp.zeros((16,), jnp.float32)
            return carry

        lax.fori_loop(0, EB, zr, 0)
        for k in range(NPT // EB):
            pltpu.sync_copy(buf, acc.at[pl.ds(s * NPT + k * EB, EB)])
        plsc.subcore_barrier()

        def blk(i, carry):
            off = base + i * EB
            pltpu.sync_copy(row_h.at[pl.ds(off, EB)], idx)
            pltpu.sync_copy(m2_h.at[pl.ds(off, EB)], buf)
            pltpu.sync_copy(buf, acc.at[idx], add=True)
            return carry

        lax.fori_loop(0, NBLK, blk, 0)
        plsc.subcore_barrier()
        pltpu.sync_copy(acc.at[pl.ds(s * NPT, NPT)],
                        out_h.at[c, pl.ds(s * NPT, NPT)])

    out = sk(m2, row)
    return out[0], out[1]


def _segsum_jnp(m2, row):
    agg = jax.ops.segment_sum(m2, row, num_segments=N_PAD)
    return agg, jnp.zeros_like(agg)


def kernel(x, pos, extra_x, edge_index, edge_attr, batch, time, params):
    f32 = jnp.float32
    lyr = params["layers"]
    w1 = [l["edge_mlp"][0]["w"] for l in lyr]
    b1 = [l["edge_mlp"][0]["b"].reshape(1, EMB) for l in lyr]
    w2 = [l["edge_mlp"][1]["w"] for l in lyr]
    b2 = [l["edge_mlp"][1]["b"].reshape(1, EMB) for l in lyr]
    wn0 = [l["node_mlp"][0]["w"] for l in lyr]
    bn0 = [l["node_mlp"][0]["b"].reshape(1, EMB) for l in lyr]
    wn1 = [l["node_mlp"][1]["w"] for l in lyr]
    bn1 = [l["node_mlp"][1]["b"].reshape(1, EMB) for l in lyr]
    wl1 = [l["ff"]["l1"]["w"] for l in lyr]
    bl1 = [l["ff"]["l1"]["b"].reshape(1, EMB) for l in lyr]
    lng = [l["ff"]["ln_g"].reshape(1, EMB) for l in lyr]
    lnb = [l["ff"]["ln_b"].reshape(1, EMB) for l in lyr]
    wl2 = [l["ff"]["l2"]["w"] for l in lyr]
    bl2 = [l["ff"]["l2"]["b"].reshape(1, EMB) for l in lyr]

    temb0, temb1 = pl.pallas_call(
        _prep_body,
        out_shape=[
            jax.ShapeDtypeStruct((G, 2 * EMB), f32),
            jax.ShapeDtypeStruct((G, 2 * EMB), f32),
        ],
    )(time,
      params["time_mlp"][0]["w"].reshape(1, -1),
      params["time_mlp"][0]["b"].reshape(1, -1),
      params["time_mlp"][1]["w"], params["time_mlp"][1]["b"].reshape(1, -1),
      lyr[0]["time_proj"]["w"], lyr[0]["time_proj"]["b"].reshape(1, -1),
      lyr[1]["time_proj"]["w"], lyr[1]["time_proj"]["b"].reshape(1, -1))

    x_p = _pad_rows(x, N_PAD)
    ex_p = _pad_rows(extra_x, N_PAD)
    pos_p = _pad_rows(pos, N_PAD)
    batch_p = _pad_rows(batch.reshape(N, 1), N_PAD)
    npad = E_PAD - E
    fill = N + (jnp.arange(npad, dtype=jnp.int32) % (N_PAD - N))
    row_p = jnp.concatenate([edge_index[0], fill])
    col_p = jnp.concatenate([edge_index[1], fill])
    eattr_p = _pad_rows(edge_attr, E_PAD)

    grid_n = (N_PAD // BN,)
    feats0, tab0 = pl.pallas_call(
        _pre_body,
        grid=grid_n,
        in_specs=[_blk((BN, 12)), _blk((BN, 3)), _blk((BN, 3)),
                  _full(params["hyp_w"])],
        out_specs=[_blk((BN, EMB)), _blk((BN, TBL))],
        out_shape=[
            jax.ShapeDtypeStruct((N_PAD, EMB), f32),
            jax.ShapeDtypeStruct((N_PAD, TBL), f32),
        ],
    )(x_p, ex_p, pos_p, params["hyp_w"])

    grid_e = (E_PAD // BE,)

    def edge_mlp(gpair, s_e, we, be, w1_, b1_, w2_, b2_):
        ds = s_e.shape[1]
        return pl.pallas_call(
            _edge_body,
            grid=grid_e,
            in_specs=[pl.BlockSpec((1, BE, EMB), lambda i: (0, i, 0)),
                      pl.BlockSpec((1, BE, EMB), lambda i: (1, i, 0)),
                      pl.BlockSpec((1, BE, EMB), lambda i: (2, i, 0)),
                      _blk((BE, ds)), _full(we), _full(be),
                      _full(w1_), _full(b1_), _full(w2_), _full(b2_)],
            out_specs=_blk((BE, EMB)),
            out_shape=jax.ShapeDtypeStruct((E_PAD, EMB), f32),
        )(gpair, gpair, gpair, s_e, we, be, w1_, b1_, w2_, b2_)

    def node_update(li, feats, agga, aggb, coors, temb, final):
        body = functools.partial(_node_body, final=final)
        common = [feats, agga, aggb, coors, batch_p, temb,
                  wn0[li], bn0[li], wn1[li], bn1[li], wl1[li], bl1[li],
                  lng[li], lnb[li], wl2[li], bl2[li]]
        common_specs = [_blk((BN, EMB)), _blk((BN, EMB)), _blk((BN, EMB)),
                        _blk((BN, 3)), _blk((BN, 1)), _full(temb)] + \
            [_full(a) for a in common[6:]]
        if final:
            wlin = params["lin"]["w"]
            blin = params["lin"]["b"].reshape(1, -1)
            return pl.pallas_call(
                body, grid=grid_n,
                in_specs=common_specs + [_full(wlin), _full(blin)],
                out_specs=_blk((BN, 4)),
                out_shape=jax.ShapeDtypeStruct((N_PAD, 4), f32),
            )(*common, wlin, blin)
        return pl.pallas_call(
            body, grid=grid_n,
            in_specs=common_specs,
            out_specs=[_blk((BN, EMB)), _blk((BN, 3)), _blk((BN, TBL))],
            out_shape=[
                jax.ShapeDtypeStruct((N_PAD, EMB), f32),
                jax.ShapeDtypeStruct((N_PAD, 3), f32),
                jax.ShapeDtypeStruct((N_PAD, TBL), f32),
            ])(*common)

    gp0 = _gather_pairs(tab0, row_p, col_p)
    m2_0 = edge_mlp(gp0, eattr_p, params["edge_enc"]["w"],
                    params["edge_enc"]["b"].reshape(1, -1),
                    w1[0], b1[0], w2[0], b2[0])
    agg0a, agg0b = _segsum(m2_0, row_p)
    feats1, corr1, tab1 = node_update(
        0, feats0, agg0a, agg0b, pos_p, temb0, final=False)

    gp1 = _gather_pairs(tab1, row_p, col_p)
    m2_1 = edge_mlp(gp1, m2_0, lyr[0]["edge_upd"]["w"],
                    lyr[0]["edge_upd"]["b"].reshape(1, -1),
                    w1[1], b1[1], w2[1], b2[1])
    agg1a, agg1b = _segsum(m2_1, row_p)
    out_p = node_update(1, feats1, agg1a, agg1b, corr1, temb1, final=True)

    return out_p[:N]
